# baseline (speedup 1.0000x reference)
import functools

import jax
import jax.numpy as jnp
from jax.experimental import pallas as pl

NUM_EXPERTS = 8
TOPK = 2
HIDDEN = 1024
T_BLK = 2048


def _router_block(w_ref, h_ref, probs_ref, idx_ref):
    h = h_ref[...]
    w = w_ref[...]
    logits = jax.lax.dot_general(
        h, w, (((1,), (1,)), ((), ())),
        preferred_element_type=jnp.float32,
    )

    e_iota = jax.lax.broadcasted_iota(jnp.int32, logits.shape, 1)
    m1 = jnp.max(logits, axis=-1)
    i1 = jnp.argmax(logits, axis=-1).astype(jnp.int32)
    masked = jnp.where(e_iota == i1[:, None], -jnp.inf, logits)
    m2 = jnp.max(masked, axis=-1)
    i2 = jnp.argmax(masked, axis=-1).astype(jnp.int32)

    ed = jnp.exp(m2 - m1)
    denom = 1.0 + ed
    p1 = 1.0 / denom
    p2 = ed / denom

    probs_ref[...] = jnp.stack([p1, p2], axis=-1)
    idx_ref[...] = jnp.stack([i1, i2], axis=-1)


@jax.jit
def kernel(hidden_states, weight):
    S, B, H = hidden_states.shape
    T = S * B
    h = hidden_states.reshape(T, H)
    grid = (T // T_BLK,)
    probs, idx = pl.pallas_call(
        _router_block,
        grid=grid,
        in_specs=[
            pl.BlockSpec((NUM_EXPERTS, H), lambda i: (0, 0)),
            pl.BlockSpec((T_BLK, H), lambda i: (i, 0)),
        ],
        out_specs=[
            pl.BlockSpec((T_BLK, TOPK), lambda i: (i, 0)),
            pl.BlockSpec((T_BLK, TOPK), lambda i: (i, 0)),
        ],
        out_shape=[
            jax.ShapeDtypeStruct((T, TOPK), jnp.float32),
            jax.ShapeDtypeStruct((T, TOPK), jnp.int32),
        ],
    )(weight, h)
    return (probs, idx)


# speedup vs baseline: 1.0052x; 1.0052x over previous
import functools

import jax
import jax.numpy as jnp
from jax.experimental import pallas as pl

NUM_EXPERTS = 8
TOPK = 2
HIDDEN = 1024
T_BLK = 4096


def _router_block(w_ref, h_ref, probs_ref, idx_ref):
    h = h_ref[...]
    w = w_ref[...]
    logits = jax.lax.dot_general(
        h, w, (((1,), (1,)), ((), ())),
        preferred_element_type=jnp.float32,
    )

    e_iota = jax.lax.broadcasted_iota(jnp.int32, logits.shape, 1)
    m1 = jnp.max(logits, axis=-1)
    i1 = jnp.argmax(logits, axis=-1).astype(jnp.int32)
    masked = jnp.where(e_iota == i1[:, None], -jnp.inf, logits)
    m2 = jnp.max(masked, axis=-1)
    i2 = jnp.argmax(masked, axis=-1).astype(jnp.int32)

    ed = jnp.exp(m2 - m1)
    denom = 1.0 + ed
    p1 = 1.0 / denom
    p2 = ed / denom

    probs_ref[...] = jnp.stack([p1, p2], axis=-1)
    idx_ref[...] = jnp.stack([i1, i2], axis=-1)


@jax.jit
def kernel(hidden_states, weight):
    S, B, H = hidden_states.shape
    T = S * B
    h = hidden_states.reshape(T, H)
    grid = (T // T_BLK,)
    probs, idx = pl.pallas_call(
        _router_block,
        grid=grid,
        in_specs=[
            pl.BlockSpec((NUM_EXPERTS, H), lambda i: (0, 0)),
            pl.BlockSpec((T_BLK, H), lambda i: (i, 0)),
        ],
        out_specs=[
            pl.BlockSpec((T_BLK, TOPK), lambda i: (i, 0)),
            pl.BlockSpec((T_BLK, TOPK), lambda i: (i, 0)),
        ],
        out_shape=[
            jax.ShapeDtypeStruct((T, TOPK), jnp.float32),
            jax.ShapeDtypeStruct((T, TOPK), jnp.int32),
        ],
    )(weight, h)
    return (probs, idx)
